# Initial kernel scaffold; baseline (speedup 1.0000x reference)
import jax
import jax.numpy as jnp
from jax.experimental import pallas as pl


def kernel(f2, f3, f4, f5, proposals):
    raise NotImplementedError("write your pallas kernel here")


# speedup vs baseline: 7.1722x; 7.1722x over previous
import jax
import jax.numpy as jnp
from jax import lax
from jax.experimental import pallas as pl
from jax.experimental.pallas import tpu as pltpu
from jax.experimental.pallas import tpu_sc as plsc

CROP = 7
NSAMP = CROP * CROP
NROWS = 4 * NSAMP
HALF = NROWS // 2
HALF_PAD = 104
C = 256
N_ROI = 1000
NW = 32
ROIS_PER_W = 32


def _sc_body(table, prop, out, prop_v, idx_v, w_v, rows_v, out_v, sem):
    wid = lax.axis_index("s") * 2 + lax.axis_index("c")
    base_roi = wid * ROIS_PER_W
    pltpu.sync_copy(prop.at[pl.ds(base_roi * 7, ROIS_PER_W * 7)], prop_v)

    iota = lax.iota(jnp.int32, 16)
    zeros16 = jnp.zeros((16,), dtype=jnp.int32)
    idx_v[0, pl.ds(88, 16)] = zeros16
    idx_v[1, pl.ds(88, 16)] = zeros16

    def roi_body(r, carry):
        roi = base_roi + r

        @pl.when(roi < N_ROI)
        def _():
            r7 = jnp.full((16,), r * 7, dtype=jnp.int32)

            def col(j):
                return plsc.load_gather(prop_v, [r7 + j])

            x0, y0, x1, y1 = col(1), col(2), col(3), col(4)
            area = (x1 - x0) * (y1 - y0)
            one = jnp.full((16,), 1, dtype=jnp.int32)
            zero = jnp.full((16,), 0, dtype=jnp.int32)
            lvl = (jnp.where(area > 2304.0, one, zero)
                   + jnp.where(area > 9216.0, one, zero)
                   + jnp.where(area > 36864.0, one, zero))
            scale = jnp.where(lvl == 0, 0.25,
                              jnp.where(lvl == 1, 0.125,
                                        jnp.where(lvl == 2, 0.0625, 0.03125)))
            off = jnp.where(lvl == 0, 0,
                            jnp.where(lvl == 1, 16384,
                                      jnp.where(lvl == 2, 20480, 21504))).astype(jnp.int32)
            wl = jnp.where(lvl == 0, 128,
                           jnp.where(lvl == 1, 64,
                                     jnp.where(lvl == 2, 32, 16))).astype(jnp.int32)
            x0s = x0 * scale
            y0s = y0 * scale
            bw = (x1 * scale - x0s) / 7.0
            bh = (y1 * scale - y0s) / 7.0
            wmax = wl - 1

            for j in range(4):
                p = iota + (16 * j)
                pyi = (p * 9363) >> 16
                pxi = p - pyi * 7
                yy = y0s + (pyi.astype(jnp.float32) + 0.5) * bh - 0.5
                xx = x0s + (pxi.astype(jnp.float32) + 0.5) * bw - 0.5
                yt = yy.astype(jnp.int32)
                yfi = jnp.where(yt.astype(jnp.float32) > yy, yt - 1, yt)
                xt = xx.astype(jnp.int32)
                xfi = jnp.where(xt.astype(jnp.float32) > xx, xt - 1, xt)
                ly = yy - yfi.astype(jnp.float32)
                lx = xx - xfi.astype(jnp.float32)
                hy = 1.0 - ly
                hx = 1.0 - lx
                y0c = jnp.clip(yfi, 0, wmax)
                y1c = jnp.clip(yfi + 1, 0, wmax)
                x0c = jnp.clip(xfi, 0, wmax)
                x1c = jnp.clip(xfi + 1, 0, wmax)
                b0 = off + y0c * wl
                b1 = off + y1c * wl
                idx4 = (b0 + x0c, b0 + x1c, b1 + x0c, b1 + x1c)
                w4 = (hy * hx, hy * lx, ly * hx, ly * lx)
                msk = p < NSAMP
                pos = p * 4
                for c4 in range(4):
                    posc = pos + c4
                    g = (posc * 669) >> 16
                    rem = posc - g * HALF
                    plsc.store_scatter(idx_v, [g, rem], idx4[c4], mask=msk)
                    plsc.store_scatter(w_v, [p, jnp.full((16,), c4, dtype=jnp.int32)],
                                       w4[c4], mask=msk)

            cp0 = pltpu.async_copy(table.at[idx_v.at[0]],
                                   rows_v.at[pl.ds(0, HALF_PAD)], sem)
            cp1 = pltpu.async_copy(table.at[idx_v.at[1]],
                                   rows_v.at[pl.ds(HALF_PAD, HALF_PAD)], sem)
            cp0.wait()
            cp1.wait()

            def py_body(py, c0):
                def px_body(px, c1):
                    s = py * CROP + px
                    s4 = s * 4
                    wrow = w_v[s, :]
                    w00 = wrow[0]
                    w01 = wrow[1]
                    w10 = wrow[2]
                    w11 = wrow[3]
                    pyv = jnp.full((16,), py, dtype=jnp.int32)
                    pxv = jnp.full((16,), px, dtype=jnp.int32)
                    def row(c4):
                        posc = s4 + c4
                        return posc + 6 * ((posc * 669) >> 16)
                    r0, r1, r2, r3 = row(0), row(1), row(2), row(3)
                    for cb in range(16):
                        sl = pl.ds(cb * 16, 16)
                        acc = (rows_v[r0, sl] * w00 + rows_v[r1, sl] * w01
                               + rows_v[r2, sl] * w10 + rows_v[r3, sl] * w11)
                        plsc.store_scatter(out_v, [iota + cb * 16, pyv, pxv], acc)
                    return c1
                return lax.fori_loop(0, CROP, px_body, c0)

            lax.fori_loop(0, CROP, py_body, 0)
            pltpu.sync_copy(out_v, out.at[roi])

        return carry

    lax.fori_loop(0, ROIS_PER_W, roi_body, 0)


def kernel(f2, f3, f4, f5, proposals):
    parts = [jnp.transpose(f[0], (1, 2, 0)).reshape(-1, C) for f in (f2, f3, f4, f5)]
    table = jnp.concatenate(parts, axis=0)
    prop = jnp.pad(proposals, ((0, NW * ROIS_PER_W - proposals.shape[0]), (0, 0))).reshape(-1)
    mesh = plsc.VectorSubcoreMesh(core_axis_name="c", subcore_axis_name="s")
    k = pl.kernel(
        _sc_body,
        out_type=jax.ShapeDtypeStruct((N_ROI, C, CROP, CROP), jnp.float32),
        mesh=mesh,
        scratch_types=[
            pltpu.VMEM((ROIS_PER_W * 7,), jnp.float32),
            pltpu.VMEM((2, HALF_PAD), jnp.int32),
            pltpu.VMEM((NSAMP, 16), jnp.float32),
            pltpu.VMEM((2 * HALF_PAD, C), jnp.float32),
            pltpu.VMEM((C, CROP, CROP), jnp.float32),
            pltpu.SemaphoreType.DMA,
        ],
        compiler_params=pltpu.CompilerParams(use_tc_tiling_on_sc=False,
                                             needs_layout_passes=False),
    )
    return k(table, prop)
